# Initial kernel scaffold; baseline (speedup 1.0000x reference)
import jax
import jax.numpy as jnp
from jax.experimental import pallas as pl


def kernel(features, A, W0, b0, W1, b1):
    raise NotImplementedError("write your pallas kernel here")


# speedup vs baseline: 3699.6177x; 3699.6177x over previous
import functools

import jax
import jax.numpy as jnp
from jax.experimental import pallas as pl
from jax.experimental.pallas import tpu as pltpu

N = 4096
F = 128

BI = 512
BJ = 512
BR = 512


def _deg_kernel(a_ref, dis_ref):
    i = pl.program_id(0)
    s = jnp.sum(a_ref[...].astype(jnp.float32), axis=0, keepdims=True)

    @pl.when(i == 0)
    def _():
        dis_ref[...] = s

    @pl.when(i > 0)
    def _():
        dis_ref[...] += s

    @pl.when(i == pl.num_programs(0) - 1)
    def _():
        dis_ref[...] = jax.lax.rsqrt(dis_ref[...] + 1.0)


def _mm_scale_kernel(x_ref, w_ref, dis_ref, o_ref):
    o_ref[...] = dis_ref[...] * jnp.dot(
        x_ref[...], w_ref[...], preferred_element_type=jnp.float32
    )


def _prop_kernel(a_ref, hwsj_ref, hwsi_ref, disi_ref, b_ref, o_ref, acc_ref):
    j = pl.program_id(1)

    @pl.when(j == 0)
    def _():
        acc_ref[...] = jnp.zeros_like(acc_ref)

    a = a_ref[...].astype(jnp.float32)
    acc_ref[...] += jax.lax.dot_general(
        a, hwsj_ref[...], (((0,), (0,)), ((), ())),
        preferred_element_type=jnp.float32,
    )

    @pl.when(j == pl.num_programs(1) - 1)
    def _():
        di = disi_ref[...]
        o_ref[...] = jnp.maximum(
            di * (acc_ref[...] + hwsi_ref[...]) + b_ref[...], 0.0
        )


def _compute_dis(A):
    dis_row = pl.pallas_call(
        _deg_kernel,
        grid=(N // BR,),
        in_specs=[pl.BlockSpec((BR, N), lambda i: (i, 0))],
        out_specs=pl.BlockSpec((1, N), lambda i: (0, 0)),
        out_shape=jax.ShapeDtypeStruct((1, N), jnp.float32),
    )(A)
    return dis_row.reshape(N, 1)


def _layer(A, h, W, b2d, dis_col):
    hws = pl.pallas_call(
        _mm_scale_kernel,
        grid=(N // BR,),
        in_specs=[
            pl.BlockSpec((BR, h.shape[1]), lambda i: (i, 0)),
            pl.BlockSpec((h.shape[1], F), lambda i: (0, 0)),
            pl.BlockSpec((BR, 1), lambda i: (i, 0)),
        ],
        out_specs=pl.BlockSpec((BR, F), lambda i: (i, 0)),
        out_shape=jax.ShapeDtypeStruct((N, F), jnp.float32),
    )(h, W, dis_col)

    out = pl.pallas_call(
        _prop_kernel,
        grid=(N // BI, N // BJ),
        in_specs=[
            pl.BlockSpec((BJ, BI), lambda i, j: (j, i)),
            pl.BlockSpec((BJ, F), lambda i, j: (j, 0)),
            pl.BlockSpec((BI, F), lambda i, j: (i, 0)),
            pl.BlockSpec((BI, 1), lambda i, j: (i, 0)),
            pl.BlockSpec((1, F), lambda i, j: (0, 0)),
        ],
        out_specs=pl.BlockSpec((BI, F), lambda i, j: (i, 0)),
        out_shape=jax.ShapeDtypeStruct((N, F), jnp.float32),
        scratch_shapes=[pltpu.VMEM((BI, F), jnp.float32)],
        compiler_params=pltpu.CompilerParams(
            dimension_semantics=("parallel", "arbitrary"),
        ),
    )(A, hws, hws, dis_col, b2d)
    return out


def kernel(features, A, W0, b0, W1, b1):
    dis_col = _compute_dis(A)
    h1 = _layer(A, features, W0, b0.reshape(1, F), dis_col)
    h2 = _layer(A, h1, W1, b1.reshape(1, F), dis_col)
    return h2


# speedup vs baseline: 4202.2196x; 1.1359x over previous
import functools

import jax
import jax.numpy as jnp
from jax.experimental import pallas as pl
from jax.experimental.pallas import tpu as pltpu

N = 4096
F = 128

BI = 512
BJ = 512
BR = 512


def _prep_kernel(a_ref, dis_ref, a8_ref):
    i = pl.program_id(0)
    a = a_ref[...]
    a8_ref[...] = a.astype(jnp.int8)
    s = jnp.sum(a.astype(jnp.float32), axis=0, keepdims=True)

    @pl.when(i == 0)
    def _():
        dis_ref[...] = s

    @pl.when(i > 0)
    def _():
        dis_ref[...] += s

    @pl.when(i == pl.num_programs(0) - 1)
    def _():
        dis_ref[...] = jax.lax.rsqrt(dis_ref[...] + 1.0)


def _mm_scale_kernel(x_ref, w_ref, dis_ref, o_ref):
    o_ref[...] = (
        dis_ref[...]
        * jnp.dot(x_ref[...], w_ref[...], preferred_element_type=jnp.float32)
    ).astype(jnp.bfloat16)


def _prop_kernel(a8_ref, hwsj_ref, hwsi_ref, disi_ref, b_ref, o_ref, acc_ref):
    j = pl.program_id(1)

    @pl.when(j == 0)
    def _():
        acc_ref[...] = jnp.zeros_like(acc_ref)

    a = a8_ref[...].astype(jnp.bfloat16)
    acc_ref[...] += jax.lax.dot_general(
        a, hwsj_ref[...], (((0,), (0,)), ((), ())),
        preferred_element_type=jnp.float32,
    )

    @pl.when(j == pl.num_programs(1) - 1)
    def _():
        di = disi_ref[...]
        o_ref[...] = jnp.maximum(
            di * (acc_ref[...] + hwsi_ref[...].astype(jnp.float32))
            + b_ref[...],
            0.0,
        )


def _prep(A):
    dis_row, a8 = pl.pallas_call(
        _prep_kernel,
        grid=(N // BR,),
        in_specs=[pl.BlockSpec((BR, N), lambda i: (i, 0))],
        out_specs=[
            pl.BlockSpec((1, N), lambda i: (0, 0)),
            pl.BlockSpec((BR, N), lambda i: (i, 0)),
        ],
        out_shape=[
            jax.ShapeDtypeStruct((1, N), jnp.float32),
            jax.ShapeDtypeStruct((N, N), jnp.int8),
        ],
    )(A)
    return dis_row.reshape(N, 1), a8


def _layer(a8, h, W, b2d, dis_col):
    hws = pl.pallas_call(
        _mm_scale_kernel,
        grid=(N // BR,),
        in_specs=[
            pl.BlockSpec((BR, h.shape[1]), lambda i: (i, 0)),
            pl.BlockSpec((h.shape[1], F), lambda i: (0, 0)),
            pl.BlockSpec((BR, 1), lambda i: (i, 0)),
        ],
        out_specs=pl.BlockSpec((BR, F), lambda i: (i, 0)),
        out_shape=jax.ShapeDtypeStruct((N, F), jnp.bfloat16),
    )(h, W, dis_col)

    out = pl.pallas_call(
        _prop_kernel,
        grid=(N // BI, N // BJ),
        in_specs=[
            pl.BlockSpec((BJ, BI), lambda i, j: (j, i)),
            pl.BlockSpec((BJ, F), lambda i, j: (j, 0)),
            pl.BlockSpec((BI, F), lambda i, j: (i, 0)),
            pl.BlockSpec((BI, 1), lambda i, j: (i, 0)),
            pl.BlockSpec((1, F), lambda i, j: (0, 0)),
        ],
        out_specs=pl.BlockSpec((BI, F), lambda i, j: (i, 0)),
        out_shape=jax.ShapeDtypeStruct((N, F), jnp.float32),
        scratch_shapes=[pltpu.VMEM((BI, F), jnp.float32)],
        compiler_params=pltpu.CompilerParams(
            dimension_semantics=("parallel", "arbitrary"),
        ),
    )(a8, hws, hws, dis_col, b2d)
    return out


def kernel(features, A, W0, b0, W1, b1):
    dis_col, a8 = _prep(A)
    h1 = _layer(a8, features, W0, b0.reshape(1, F), dis_col)
    h2 = _layer(a8, h1, W1, b1.reshape(1, F), dis_col)
    return h2


# speedup vs baseline: 5638.1666x; 1.3417x over previous
import functools

import jax
import jax.numpy as jnp
from jax.experimental import pallas as pl
from jax.experimental.pallas import tpu as pltpu

N = 4096
F = 128

BI = 512
BJ = 1024
BR = 512


def _prep_kernel(a_ref, dis_ref, a8_ref):
    i = pl.program_id(0)
    a = a_ref[...]
    a8_ref[...] = a.astype(jnp.int8)
    s = jnp.sum(a.astype(jnp.float32), axis=0, keepdims=True)

    @pl.when(i == 0)
    def _():
        dis_ref[...] = s

    @pl.when(i > 0)
    def _():
        dis_ref[...] += s

    @pl.when(i == pl.num_programs(0) - 1)
    def _():
        dis_ref[...] = jax.lax.rsqrt(dis_ref[...] + 1.0)


def _mm_scale_kernel(x_ref, w_ref, dis_ref, o_ref):
    o_ref[...] = (
        dis_ref[...]
        * jnp.dot(x_ref[...], w_ref[...], preferred_element_type=jnp.float32)
    ).astype(jnp.bfloat16)


def _prop_kernel(a8_ref, hwsj_ref, hwsi_ref, disi_ref, b_ref, o_ref, acc_ref):
    j = pl.program_id(1)

    @pl.when(j == 0)
    def _():
        acc_ref[...] = jnp.zeros_like(acc_ref)

    a = a8_ref[...].astype(jnp.bfloat16)
    acc_ref[...] += jax.lax.dot_general(
        a, hwsj_ref[...], (((0,), (0,)), ((), ())),
        preferred_element_type=jnp.float32,
    )

    @pl.when(j == pl.num_programs(1) - 1)
    def _():
        di = disi_ref[...]
        o_ref[...] = jnp.maximum(
            di * (acc_ref[...] + hwsi_ref[...].astype(jnp.float32))
            + b_ref[...],
            0.0,
        )


def _prep(A):
    dis_row, a8 = pl.pallas_call(
        _prep_kernel,
        grid=(N // BR,),
        in_specs=[pl.BlockSpec((BR, N), lambda i: (i, 0))],
        out_specs=[
            pl.BlockSpec((1, N), lambda i: (0, 0)),
            pl.BlockSpec((BR, N), lambda i: (i, 0)),
        ],
        out_shape=[
            jax.ShapeDtypeStruct((1, N), jnp.float32),
            jax.ShapeDtypeStruct((N, N), jnp.int8),
        ],
    )(A)
    return dis_row.reshape(N, 1), a8


def _layer(a8, h, W, b2d, dis_col):
    hws = pl.pallas_call(
        _mm_scale_kernel,
        grid=(N // BR,),
        in_specs=[
            pl.BlockSpec((BR, h.shape[1]), lambda i: (i, 0)),
            pl.BlockSpec((h.shape[1], F), lambda i: (0, 0)),
            pl.BlockSpec((BR, 1), lambda i: (i, 0)),
        ],
        out_specs=pl.BlockSpec((BR, F), lambda i: (i, 0)),
        out_shape=jax.ShapeDtypeStruct((N, F), jnp.bfloat16),
    )(h, W, dis_col)

    out = pl.pallas_call(
        _prop_kernel,
        grid=(N // BI, N // BJ),
        in_specs=[
            pl.BlockSpec((BJ, BI), lambda i, j: (j, i)),
            pl.BlockSpec((BJ, F), lambda i, j: (j, 0)),
            pl.BlockSpec((BI, F), lambda i, j: (i, 0)),
            pl.BlockSpec((BI, 1), lambda i, j: (i, 0)),
            pl.BlockSpec((1, F), lambda i, j: (0, 0)),
        ],
        out_specs=pl.BlockSpec((BI, F), lambda i, j: (i, 0)),
        out_shape=jax.ShapeDtypeStruct((N, F), jnp.float32),
        scratch_shapes=[pltpu.VMEM((BI, F), jnp.float32)],
        compiler_params=pltpu.CompilerParams(
            dimension_semantics=("parallel", "arbitrary"),
        ),
    )(a8, hws, hws, dis_col, b2d)
    return out


def kernel(features, A, W0, b0, W1, b1):
    dis_col, a8 = _prep(A)
    h1 = _layer(a8, features, W0, b0.reshape(1, F), dis_col)
    h2 = _layer(a8, h1, W1, b1.reshape(1, F), dis_col)
    return h2


# speedup vs baseline: 6830.8437x; 1.2115x over previous
import functools

import jax
import jax.numpy as jnp
from jax.experimental import pallas as pl
from jax.experimental.pallas import tpu as pltpu

N = 4096
F = 128

BI = 512
BJ = 2048
BR = 512


def _prep_kernel(a_ref, dis_ref, a8_ref):
    i = pl.program_id(0)
    a = a_ref[...]
    a8_ref[...] = a.astype(jnp.int8)
    s = jnp.sum(a.astype(jnp.float32), axis=0, keepdims=True)

    @pl.when(i == 0)
    def _():
        dis_ref[...] = s

    @pl.when(i > 0)
    def _():
        dis_ref[...] += s

    @pl.when(i == pl.num_programs(0) - 1)
    def _():
        dis_ref[...] = jax.lax.rsqrt(dis_ref[...] + 1.0)


def _mm_scale_kernel(x_ref, w_ref, dis_ref, o_ref):
    o_ref[...] = (
        dis_ref[...]
        * jnp.dot(x_ref[...], w_ref[...], preferred_element_type=jnp.float32)
    ).astype(jnp.bfloat16)


def _prop_kernel(a8_ref, hwsj_ref, hwsi_ref, disi_ref, b_ref, o_ref, acc_ref):
    j = pl.program_id(1)

    @pl.when(j == 0)
    def _():
        acc_ref[...] = jnp.zeros_like(acc_ref)

    a = a8_ref[...].astype(jnp.bfloat16)
    acc_ref[...] += jax.lax.dot_general(
        a, hwsj_ref[...], (((0,), (0,)), ((), ())),
        preferred_element_type=jnp.float32,
    )

    @pl.when(j == pl.num_programs(1) - 1)
    def _():
        di = disi_ref[...]
        o_ref[...] = jnp.maximum(
            di * (acc_ref[...] + hwsi_ref[...].astype(jnp.float32))
            + b_ref[...],
            0.0,
        )


def _prep(A):
    dis_row, a8 = pl.pallas_call(
        _prep_kernel,
        grid=(N // BR,),
        in_specs=[pl.BlockSpec((BR, N), lambda i: (i, 0))],
        out_specs=[
            pl.BlockSpec((1, N), lambda i: (0, 0)),
            pl.BlockSpec((BR, N), lambda i: (i, 0)),
        ],
        out_shape=[
            jax.ShapeDtypeStruct((1, N), jnp.float32),
            jax.ShapeDtypeStruct((N, N), jnp.int8),
        ],
    )(A)
    return dis_row.reshape(N, 1), a8


def _layer(a8, h, W, b2d, dis_col):
    hws = pl.pallas_call(
        _mm_scale_kernel,
        grid=(N // BR,),
        in_specs=[
            pl.BlockSpec((BR, h.shape[1]), lambda i: (i, 0)),
            pl.BlockSpec((h.shape[1], F), lambda i: (0, 0)),
            pl.BlockSpec((BR, 1), lambda i: (i, 0)),
        ],
        out_specs=pl.BlockSpec((BR, F), lambda i: (i, 0)),
        out_shape=jax.ShapeDtypeStruct((N, F), jnp.bfloat16),
    )(h, W, dis_col)

    out = pl.pallas_call(
        _prop_kernel,
        grid=(N // BI, N // BJ),
        in_specs=[
            pl.BlockSpec((BJ, BI), lambda i, j: (j, i)),
            pl.BlockSpec((BJ, F), lambda i, j: (j, 0)),
            pl.BlockSpec((BI, F), lambda i, j: (i, 0)),
            pl.BlockSpec((BI, 1), lambda i, j: (i, 0)),
            pl.BlockSpec((1, F), lambda i, j: (0, 0)),
        ],
        out_specs=pl.BlockSpec((BI, F), lambda i, j: (i, 0)),
        out_shape=jax.ShapeDtypeStruct((N, F), jnp.float32),
        scratch_shapes=[pltpu.VMEM((BI, F), jnp.float32)],
        compiler_params=pltpu.CompilerParams(
            dimension_semantics=("parallel", "arbitrary"),
        ),
    )(a8, hws, hws, dis_col, b2d)
    return out


def kernel(features, A, W0, b0, W1, b1):
    dis_col, a8 = _prep(A)
    h1 = _layer(a8, features, W0, b0.reshape(1, F), dis_col)
    h2 = _layer(a8, h1, W1, b1.reshape(1, F), dis_col)
    return h2


# speedup vs baseline: 7905.6525x; 1.1573x over previous
import functools

import jax
import jax.numpy as jnp
from jax.experimental import pallas as pl
from jax.experimental.pallas import tpu as pltpu

N = 4096
F = 128

BI = 512
BJ = 4096
BR = 512


def _prep_kernel(a_ref, dis_ref, a8_ref):
    i = pl.program_id(0)
    a = a_ref[...]
    a8_ref[...] = a.astype(jnp.int8)
    s = jnp.sum(a.astype(jnp.float32), axis=0, keepdims=True)

    @pl.when(i == 0)
    def _():
        dis_ref[...] = s

    @pl.when(i > 0)
    def _():
        dis_ref[...] += s

    @pl.when(i == pl.num_programs(0) - 1)
    def _():
        dis_ref[...] = jax.lax.rsqrt(dis_ref[...] + 1.0)


def _mm_scale_kernel(x_ref, w_ref, dis_ref, o_ref):
    o_ref[...] = (
        dis_ref[...]
        * jnp.dot(x_ref[...], w_ref[...], preferred_element_type=jnp.float32)
    ).astype(jnp.bfloat16)


def _prop_kernel(a8_ref, hwsj_ref, hwsi_ref, disi_ref, b_ref, o_ref, acc_ref):
    j = pl.program_id(1)

    @pl.when(j == 0)
    def _():
        acc_ref[...] = jnp.zeros_like(acc_ref)

    a = a8_ref[...].astype(jnp.bfloat16)
    acc_ref[...] += jax.lax.dot_general(
        a, hwsj_ref[...], (((0,), (0,)), ((), ())),
        preferred_element_type=jnp.float32,
    )

    @pl.when(j == pl.num_programs(1) - 1)
    def _():
        di = disi_ref[...]
        o_ref[...] = jnp.maximum(
            di * (acc_ref[...] + hwsi_ref[...].astype(jnp.float32))
            + b_ref[...],
            0.0,
        )


def _prep(A):
    dis_row, a8 = pl.pallas_call(
        _prep_kernel,
        grid=(N // BR,),
        in_specs=[pl.BlockSpec((BR, N), lambda i: (i, 0))],
        out_specs=[
            pl.BlockSpec((1, N), lambda i: (0, 0)),
            pl.BlockSpec((BR, N), lambda i: (i, 0)),
        ],
        out_shape=[
            jax.ShapeDtypeStruct((1, N), jnp.float32),
            jax.ShapeDtypeStruct((N, N), jnp.int8),
        ],
    )(A)
    return dis_row.reshape(N, 1), a8


def _layer(a8, h, W, b2d, dis_col):
    hws = pl.pallas_call(
        _mm_scale_kernel,
        grid=(N // BR,),
        in_specs=[
            pl.BlockSpec((BR, h.shape[1]), lambda i: (i, 0)),
            pl.BlockSpec((h.shape[1], F), lambda i: (0, 0)),
            pl.BlockSpec((BR, 1), lambda i: (i, 0)),
        ],
        out_specs=pl.BlockSpec((BR, F), lambda i: (i, 0)),
        out_shape=jax.ShapeDtypeStruct((N, F), jnp.bfloat16),
    )(h, W, dis_col)

    out = pl.pallas_call(
        _prop_kernel,
        grid=(N // BI, N // BJ),
        in_specs=[
            pl.BlockSpec((BJ, BI), lambda i, j: (j, i)),
            pl.BlockSpec((BJ, F), lambda i, j: (j, 0)),
            pl.BlockSpec((BI, F), lambda i, j: (i, 0)),
            pl.BlockSpec((BI, 1), lambda i, j: (i, 0)),
            pl.BlockSpec((1, F), lambda i, j: (0, 0)),
        ],
        out_specs=pl.BlockSpec((BI, F), lambda i, j: (i, 0)),
        out_shape=jax.ShapeDtypeStruct((N, F), jnp.float32),
        scratch_shapes=[pltpu.VMEM((BI, F), jnp.float32)],
        compiler_params=pltpu.CompilerParams(
            dimension_semantics=("parallel", "arbitrary"),
        ),
    )(a8, hws, hws, dis_col, b2d)
    return out


def kernel(features, A, W0, b0, W1, b1):
    dis_col, a8 = _prep(A)
    h1 = _layer(a8, features, W0, b0.reshape(1, F), dis_col)
    h2 = _layer(a8, h1, W1, b1.reshape(1, F), dis_col)
    return h2


# speedup vs baseline: 8036.3758x; 1.0165x over previous
import functools

import jax
import jax.numpy as jnp
from jax.experimental import pallas as pl
from jax.experimental.pallas import tpu as pltpu

N = 4096
F = 128

BI = 1024
BJ = 4096
BR = 512


def _prep_kernel(a_ref, dis_ref, a8_ref):
    i = pl.program_id(0)
    a = a_ref[...]
    a8_ref[...] = a.astype(jnp.int8)
    s = jnp.sum(a.astype(jnp.float32), axis=0, keepdims=True)

    @pl.when(i == 0)
    def _():
        dis_ref[...] = s

    @pl.when(i > 0)
    def _():
        dis_ref[...] += s

    @pl.when(i == pl.num_programs(0) - 1)
    def _():
        dis_ref[...] = jax.lax.rsqrt(dis_ref[...] + 1.0)


def _mm_scale_kernel(x_ref, w_ref, dis_ref, o_ref):
    o_ref[...] = (
        dis_ref[...]
        * jnp.dot(x_ref[...], w_ref[...], preferred_element_type=jnp.float32)
    ).astype(jnp.bfloat16)


def _prop_kernel(a8_ref, hwsj_ref, hwsi_ref, disi_ref, b_ref, o_ref, acc_ref):
    j = pl.program_id(1)

    @pl.when(j == 0)
    def _():
        acc_ref[...] = jnp.zeros_like(acc_ref)

    a = a8_ref[...].astype(jnp.bfloat16)
    acc_ref[...] += jax.lax.dot_general(
        a, hwsj_ref[...], (((0,), (0,)), ((), ())),
        preferred_element_type=jnp.float32,
    )

    @pl.when(j == pl.num_programs(1) - 1)
    def _():
        di = disi_ref[...]
        o_ref[...] = jnp.maximum(
            di * (acc_ref[...] + hwsi_ref[...].astype(jnp.float32))
            + b_ref[...],
            0.0,
        )


def _prep(A):
    dis_row, a8 = pl.pallas_call(
        _prep_kernel,
        grid=(N // BR,),
        in_specs=[pl.BlockSpec((BR, N), lambda i: (i, 0))],
        out_specs=[
            pl.BlockSpec((1, N), lambda i: (0, 0)),
            pl.BlockSpec((BR, N), lambda i: (i, 0)),
        ],
        out_shape=[
            jax.ShapeDtypeStruct((1, N), jnp.float32),
            jax.ShapeDtypeStruct((N, N), jnp.int8),
        ],
    )(A)
    return dis_row.reshape(N, 1), a8


def _layer(a8, h, W, b2d, dis_col):
    hws = pl.pallas_call(
        _mm_scale_kernel,
        grid=(N // BR,),
        in_specs=[
            pl.BlockSpec((BR, h.shape[1]), lambda i: (i, 0)),
            pl.BlockSpec((h.shape[1], F), lambda i: (0, 0)),
            pl.BlockSpec((BR, 1), lambda i: (i, 0)),
        ],
        out_specs=pl.BlockSpec((BR, F), lambda i: (i, 0)),
        out_shape=jax.ShapeDtypeStruct((N, F), jnp.bfloat16),
    )(h, W, dis_col)

    out = pl.pallas_call(
        _prop_kernel,
        grid=(N // BI, N // BJ),
        in_specs=[
            pl.BlockSpec((BJ, BI), lambda i, j: (j, i)),
            pl.BlockSpec((BJ, F), lambda i, j: (j, 0)),
            pl.BlockSpec((BI, F), lambda i, j: (i, 0)),
            pl.BlockSpec((BI, 1), lambda i, j: (i, 0)),
            pl.BlockSpec((1, F), lambda i, j: (0, 0)),
        ],
        out_specs=pl.BlockSpec((BI, F), lambda i, j: (i, 0)),
        out_shape=jax.ShapeDtypeStruct((N, F), jnp.float32),
        scratch_shapes=[pltpu.VMEM((BI, F), jnp.float32)],
        compiler_params=pltpu.CompilerParams(
            dimension_semantics=("parallel", "arbitrary"),
        ),
    )(a8, hws, hws, dis_col, b2d)
    return out


def kernel(features, A, W0, b0, W1, b1):
    dis_col, a8 = _prep(A)
    h1 = _layer(a8, features, W0, b0.reshape(1, F), dis_col)
    h2 = _layer(a8, h1, W1, b1.reshape(1, F), dis_col)
    return h2


# speedup vs baseline: 9376.7478x; 1.1668x over previous
import functools

import jax
import jax.numpy as jnp
from jax.experimental import pallas as pl
from jax.experimental.pallas import tpu as pltpu

N = 4096
F = 128

BI = 1024
BR = 512


def _prep_kernel(a_ref, dis_ref, a8_ref):
    i = pl.program_id(0)
    a = a_ref[...]
    a8_ref[...] = a.astype(jnp.int8)
    s = jnp.sum(a.astype(jnp.float32), axis=0, keepdims=True)

    @pl.when(i == 0)
    def _():
        dis_ref[...] = s

    @pl.when(i > 0)
    def _():
        dis_ref[...] += s

    @pl.when(i == pl.num_programs(0) - 1)
    def _():
        dis_ref[...] = jax.lax.rsqrt(dis_ref[...] + 1.0)


def _gcn_kernel(a8_ref, x_ref, w_ref, b_ref, dis_ref, o_ref, hws_ref, h1_ref):
    l = pl.program_id(0)
    i = pl.program_id(1)

    @pl.when((l == 0) & (i == 0))
    def _():
        hws_ref[...] = (
            dis_ref[...]
            * jnp.dot(x_ref[...], w_ref[0], preferred_element_type=jnp.float32)
        ).astype(jnp.bfloat16)

    @pl.when((l == 1) & (i == 0))
    def _():
        hws_ref[...] = (
            dis_ref[...]
            * jnp.dot(h1_ref[...], w_ref[0], preferred_element_type=jnp.float32)
        ).astype(jnp.bfloat16)

    a = a8_ref[...].astype(jnp.bfloat16)
    acc = jax.lax.dot_general(
        a, hws_ref[...], (((0,), (0,)), ((), ())),
        preferred_element_type=jnp.float32,
    )
    sl = pl.ds(i * BI, BI)
    di = dis_ref[sl, :]
    res = jnp.maximum(
        di * (acc + hws_ref[sl, :].astype(jnp.float32)) + b_ref[0], 0.0
    )

    @pl.when(l == 0)
    def _():
        h1_ref[sl, :] = res

    o_ref[0] = res


def kernel(features, A, W0, b0, W1, b1):
    dis_row, a8 = pl.pallas_call(
        _prep_kernel,
        grid=(N // BR,),
        in_specs=[pl.BlockSpec((BR, N), lambda i: (i, 0))],
        out_specs=[
            pl.BlockSpec((1, N), lambda i: (0, 0)),
            pl.BlockSpec((BR, N), lambda i: (i, 0)),
        ],
        out_shape=[
            jax.ShapeDtypeStruct((1, N), jnp.float32),
            jax.ShapeDtypeStruct((N, N), jnp.int8),
        ],
    )(A)
    dis_col = dis_row.reshape(N, 1)

    w_stack = jnp.stack([W0, W1])
    b_stack = jnp.stack([b0.reshape(1, F), b1.reshape(1, F)])

    out = pl.pallas_call(
        _gcn_kernel,
        grid=(2, N // BI),
        in_specs=[
            pl.BlockSpec((N, BI), lambda l, i: (0, i)),
            pl.BlockSpec((N, F), lambda l, i: (0, 0)),
            pl.BlockSpec((1, F, F), lambda l, i: (l, 0, 0)),
            pl.BlockSpec((1, 1, F), lambda l, i: (l, 0, 0)),
            pl.BlockSpec((N, 1), lambda l, i: (0, 0)),
        ],
        out_specs=pl.BlockSpec((1, BI, F), lambda l, i: (l, i, 0)),
        out_shape=jax.ShapeDtypeStruct((2, N, F), jnp.float32),
        scratch_shapes=[
            pltpu.VMEM((N, F), jnp.bfloat16),
            pltpu.VMEM((N, F), jnp.float32),
        ],
        compiler_params=pltpu.CompilerParams(
            dimension_semantics=("arbitrary", "arbitrary"),
        ),
    )(a8, features, w_stack, b_stack, dis_col)
    return out[1]
